# baseline (speedup 1.0000x reference)
import functools

import jax
import jax.numpy as jnp
from jax import lax
from jax.experimental import pallas as pl
from jax.experimental.pallas import tpu as pltpu
from jax.experimental.pallas import tpu_sc as plsc

N_PAD = 10240
G_SEG = 16
KNB = 8
KSEL = 7
DH = 336
DHP = 384
DO = 256
R_KNN = 256
C_KNN = 512
T_KNN = N_PAD // C_KNN
SLOPE = 0.01

_F32 = jnp.float32
_HIGH = lax.Precision.HIGHEST
_BIGI = 2**30


def _lrelu(v):
    return jnp.maximum(v, SLOPE * v)


def _dot(a, b):
    return lax.dot_general(a, b, (((1,), (0,)), ((), ())),
                           precision=_HIGH, preferred_element_type=_F32)


def _dot_t(a, b):
    return lax.dot_general(a, b, (((1,), (1,)), ((), ())),
                           precision=_HIGH, preferred_element_type=_F32)


def _proj_body(x_ref, w_ref, b_ref, a_ref, bo_ref, n_ref):
    x = x_ref[...]
    p = _dot(x, w_ref[...]) + b_ref[...]
    a_ref[...] = p[:, :DHP]
    bo_ref[...] = p[:, DHP:]
    n_ref[...] = jnp.sum(x * x, axis=1, keepdims=True)


def _proj(x, w, b):
    n, d = x.shape
    m = w.shape[1]
    blk = 512
    return pl.pallas_call(
        _proj_body,
        grid=(n // blk,),
        in_specs=[
            pl.BlockSpec((blk, d), lambda i: (i, 0)),
            pl.BlockSpec((d, m), lambda i: (0, 0)),
            pl.BlockSpec((1, m), lambda i: (0, 0)),
        ],
        out_specs=[
            pl.BlockSpec((blk, DHP), lambda i: (i, 0)),
            pl.BlockSpec((blk, DHP), lambda i: (i, 0)),
            pl.BlockSpec((blk, 1), lambda i: (i, 0)),
        ],
        out_shape=[
            jax.ShapeDtypeStruct((n, DHP), _F32),
            jax.ShapeDtypeStruct((n, DHP), _F32),
            jax.ShapeDtypeStruct((n, 1), _F32),
        ],
    )(x, w, b.reshape(1, m))


def _knn_body(t0_ref, nt_ref, xq_ref, xt_ref, x2q_ref, x2t_ref,
              lo_ref, hi_ref, nbr_ref, dscr):
    c = pl.program_id(0)
    t0 = t0_ref[c]
    nt = nt_ref[c]
    xq = xq_ref[...]
    x2q = x2q_ref[...]
    lo = lo_ref[...]
    hi = hi_ref[...]

    rowid = (lax.broadcasted_iota(jnp.int32, (R_KNN, 1), 0) + c * R_KNN)

    def fill(t, _):
        xct = xt_ref[t0 + t]
        x2c = x2t_ref[t0 + t]
        d2 = x2q - 2.0 * _dot(xq, xct) + x2c
        col = (lax.broadcasted_iota(jnp.int32, (R_KNN, C_KNN), 1)
               + (t0 + t) * C_KNN)
        valid = (col >= lo) & (col < hi) & (col != rowid)
        dscr[t] = jnp.where(valid, d2, jnp.inf)
        return 0

    lax.fori_loop(0, nt, fill, 0)

    prevd = jnp.full((R_KNN, 1), -jnp.inf, _F32)
    cols = []
    for _ in range(KSEL):
        def scan(t, carry, prevd=prevd):
            bmin, barg = carry
            dm = jnp.where(dscr[t] > prevd, dscr[t], jnp.inf)
            tmin = jnp.min(dm, axis=1, keepdims=True)
            col = (lax.broadcasted_iota(jnp.int32, (R_KNN, C_KNN), 1)
                   + (t0 + t) * C_KNN)
            targ = jnp.min(jnp.where(dm == tmin, col, _BIGI),
                           axis=1, keepdims=True)
            better = (tmin < bmin) | ((tmin == bmin) & (targ < barg))
            return (jnp.where(better, tmin, bmin),
                    jnp.where(better, targ, barg))

        bmin, barg = lax.fori_loop(
            0, nt, scan,
            (jnp.full((R_KNN, 1), jnp.inf, _F32),
             jnp.full((R_KNN, 1), _BIGI, jnp.int32)))
        prevd = bmin
        cols.append(barg)
    nbr = jnp.concatenate(cols, axis=1)
    nbr_ref[...] = jnp.clip(nbr, 0, N_PAD - 1)


def _knn(x, x2, rowlo, rowhi, t0s, nts):
    d = x.shape[1]
    xt_tiles = x.T.reshape(d, T_KNN, C_KNN).transpose(1, 0, 2)
    x2t_tiles = x2.reshape(T_KNN, 1, C_KNN)
    grid_spec = pltpu.PrefetchScalarGridSpec(
        num_scalar_prefetch=2,
        grid=(N_PAD // R_KNN,),
        in_specs=[
            pl.BlockSpec((R_KNN, d), lambda i, *_: (i, 0)),
            pl.BlockSpec((T_KNN, d, C_KNN), lambda i, *_: (0, 0, 0)),
            pl.BlockSpec((R_KNN, 1), lambda i, *_: (i, 0)),
            pl.BlockSpec((T_KNN, 1, C_KNN), lambda i, *_: (0, 0, 0)),
            pl.BlockSpec((R_KNN, 1), lambda i, *_: (i, 0)),
            pl.BlockSpec((R_KNN, 1), lambda i, *_: (i, 0)),
        ],
        out_specs=pl.BlockSpec((R_KNN, KSEL), lambda i, *_: (i, 0)),
        scratch_shapes=[pltpu.VMEM((T_KNN, R_KNN, C_KNN), _F32)],
    )
    return pl.pallas_call(
        _knn_body,
        grid_spec=grid_spec,
        out_shape=jax.ShapeDtypeStruct((N_PAD, KSEL), jnp.int32),
    )(t0s, nts, x, xt_tiles, x2, x2t_tiles, rowlo, rowhi)


_E_TOT = N_PAD * KSEL
_NW = 32
_E_PER_W = _E_TOT // _NW
_CH = 112
_NCH = _E_PER_W // _CH


def _sc_gather(table, idx):
    d = table.shape[1]
    mesh = plsc.VectorSubcoreMesh(core_axis_name="c", subcore_axis_name="s")

    @functools.partial(
        pl.kernel, mesh=mesh,
        out_type=jax.ShapeDtypeStruct((_E_TOT, d), _F32),
        scratch_types=[
            pltpu.VMEM((_E_PER_W,), jnp.int32),
            pltpu.VMEM((_CH, d), _F32),
            pltpu.VMEM((_CH, d), _F32),
            pltpu.SemaphoreType.DMA,
            pltpu.SemaphoreType.DMA,
            pltpu.SemaphoreType.DMA,
            pltpu.SemaphoreType.DMA,
        ],
    )
    def k(table_hbm, idx_hbm, out_hbm, idx_v, r0, r1, sg0, sg1, so0, so1):
        wid = lax.axis_index("s") * 2 + lax.axis_index("c")
        base = wid * _E_PER_W
        pltpu.sync_copy(idx_hbm.at[pl.ds(base, _E_PER_W)], idx_v)

        rows = (r0, r1)
        sg = (sg0, sg1)
        so = (so0, so1)

        def start_gather(c):
            return pltpu.async_copy(
                table_hbm.at[idx_v.at[pl.ds(c * _CH, _CH)]],
                rows[c % 2], sg[c % 2])

        def start_out(c):
            return pltpu.async_copy(
                rows[c % 2], out_hbm.at[pl.ds(base + c * _CH, _CH)],
                so[c % 2])

        gh = {0: start_gather(0)}
        oh = {}
        for c in range(_NCH):
            if c + 1 < _NCH:
                if c >= 1:
                    oh[c - 1].wait()
                gh[c + 1] = start_gather(c + 1)
            gh[c].wait()
            oh[c] = start_out(c)
        oh[_NCH - 2].wait()
        oh[_NCH - 1].wait()

    return k(table, idx)


def _edge_body(a_ref, b_ref, g_ref, w2_ref, b2_ref, o_ref):
    a = a_ref[...]
    w2 = w2_ref[...]
    acc = _dot(_lrelu(a + b_ref[...]), w2)
    for k in range(KSEL):
        h1 = _lrelu(a + g_ref[:, k, :])
        e = _dot(h1, w2)
        acc = jnp.maximum(acc, e)
    o_ref[...] = _lrelu(acc + b2_ref[...])


def _edge(a, b, gath, w2, b2):
    blk = 128
    return pl.pallas_call(
        _edge_body,
        grid=(N_PAD // blk,),
        in_specs=[
            pl.BlockSpec((blk, DHP), lambda i: (i, 0)),
            pl.BlockSpec((blk, DHP), lambda i: (i, 0)),
            pl.BlockSpec((blk, KSEL, DHP), lambda i: (i, 0, 0)),
            pl.BlockSpec((DHP, DO), lambda i: (0, 0)),
            pl.BlockSpec((1, DO), lambda i: (0, 0)),
        ],
        out_specs=pl.BlockSpec((blk, DO), lambda i: (i, 0)),
        out_shape=jax.ShapeDtypeStruct((N_PAD, DO), _F32),
    )(a, b, gath, w2, b2.reshape(1, DO))


def _m1pool_body(h_ref, m_ref, w1_ref, b1_ref, w2_ref, b2_ref, o_ref):
    @pl.when(pl.program_id(0) == 0)
    def _():
        o_ref[...] = jnp.zeros_like(o_ref)

    h1 = _lrelu(_dot(h_ref[...], w1_ref[...]) + b1_ref[...])
    h2 = _lrelu(_dot(h1, w2_ref[...]) + b2_ref[...])
    o_ref[...] += _dot(m_ref[...], h2)


def _m1pool(h, mn, w1, b1, w2, b2):
    blk = 512
    di, dh = w1.shape
    do = w2.shape[1]
    return pl.pallas_call(
        _m1pool_body,
        grid=(N_PAD // blk,),
        in_specs=[
            pl.BlockSpec((blk, di), lambda i: (i, 0)),
            pl.BlockSpec((G_SEG, blk), lambda i: (0, i)),
            pl.BlockSpec((di, dh), lambda i: (0, 0)),
            pl.BlockSpec((1, dh), lambda i: (0, 0)),
            pl.BlockSpec((dh, do), lambda i: (0, 0)),
            pl.BlockSpec((1, do), lambda i: (0, 0)),
        ],
        out_specs=pl.BlockSpec((G_SEG, do), lambda i: (0, 0)),
        out_shape=jax.ShapeDtypeStruct((G_SEG, do), _F32),
    )(h, mn, w1, b1.reshape(1, dh), w2, b2.reshape(1, do))


def _m2_body(p_ref, w1_ref, b1_ref, w2_ref, b2_ref, o_ref):
    h = _lrelu(_dot(p_ref[...], w1_ref[...]) + b1_ref[...])
    o_ref[...] = _dot(h, w2_ref[...]) + b2_ref[...]


def _m2(pooled, w1, b1, w2, b2):
    di, dh = w1.shape
    do = w2.shape[1]
    return pl.pallas_call(
        _m2_body,
        out_shape=jax.ShapeDtypeStruct((G_SEG, do), _F32),
    )(pooled, w1, b1.reshape(1, dh), w2, b2.reshape(1, do))


def _layer(xc, rowlo, rowhi, t0s, nts, p, pre):
    d = xc.shape[1]
    w1 = p[pre + "_W1"]
    zc = jnp.zeros((d, DHP - DH), _F32)
    w1a, w1b = w1[:d], w1[d:]
    wc = jnp.concatenate([w1a - w1b, zc, w1b, zc], axis=1)
    bc = jnp.concatenate([p[pre + "_b1"],
                          jnp.zeros((2 * DHP - DH,), _F32)])
    a, b, x2 = _proj(xc, wc, bc)
    nbr = _knn(xc, x2, rowlo, rowhi, t0s, nts)
    gath = _sc_gather(b, nbr.reshape(-1))
    w2p = jnp.pad(p[pre + "_W2"], ((0, DHP - DH), (0, 0)))
    return _edge(a, b, gath.reshape(N_PAD, KSEL, DHP), w2p, p[pre + "_b2"])


def kernel(x, edge_index, batch, params):
    del edge_index
    n = x.shape[0]
    pad = N_PAD - n
    xp = jnp.pad(x.astype(_F32), ((0, pad), (0, 0)))
    bp = jnp.concatenate([batch.astype(jnp.int32),
                          jnp.full((pad,), G_SEG, jnp.int32)])

    starts = jnp.searchsorted(bp, jnp.arange(G_SEG + 2, dtype=jnp.int32),
                              side="left").astype(jnp.int32)
    rowlo = starts[bp][:, None]
    rowhi = starts[bp + 1][:, None]
    b2d = bp.reshape(N_PAD // R_KNN, R_KNN)
    g0, g1 = b2d[:, 0], b2d[:, -1]
    col_lo = starts[g0]
    col_hi = starts[g1 + 1]
    t0s = col_lo // C_KNN
    nts = (col_hi + C_KNN - 1) // C_KNN - t0s

    p = params
    x1 = _layer(xp, rowlo, rowhi, t0s, nts, p, "c1")
    x2 = _layer(x1, rowlo, rowhi, t0s, nts, p, "c2")
    x3 = _layer(x2, rowlo, rowhi, t0s, nts, p, "c3")
    x4 = _layer(x3, rowlo, rowhi, t0s, nts, p, "c4")

    h = jnp.concatenate([xp, x1, x2, x3, x4], axis=1)
    m = (bp[None, :] == jnp.arange(G_SEG, dtype=jnp.int32)[:, None]
         ).astype(_F32)
    cnt = jnp.sum(m, axis=1)
    mn = m / jnp.maximum(cnt, 1.0)[:, None]
    pooled = _m1pool(h, mn, p["m1_W1"], p["m1_b1"], p["m1_W2"], p["m1_b2"])
    return _m2(pooled, p["m2_W1"], p["m2_b1"], p["m2_W2"], p["m2_b2"])


# speedup vs baseline: 1.1366x; 1.1366x over previous
import functools

import jax
import jax.numpy as jnp
from jax import lax
from jax.experimental import pallas as pl
from jax.experimental.pallas import tpu as pltpu
from jax.experimental.pallas import tpu_sc as plsc

N_PAD = 10240
G_SEG = 16
KNB = 8
KSEL = 7
DH = 336
DHP = 384
DO = 256
R_KNN = 256
C_KNN = 512
T_KNN = N_PAD // C_KNN
SLOPE = 0.01

_F32 = jnp.float32
_HIGH = lax.Precision.HIGHEST
_BIGI = 2**30


def _lrelu(v):
    return jnp.maximum(v, SLOPE * v)


def _dot(a, b):
    return lax.dot_general(a, b, (((1,), (0,)), ((), ())),
                           precision=_HIGH, preferred_element_type=_F32)


def _dot_t(a, b):
    return lax.dot_general(a, b, (((1,), (1,)), ((), ())),
                           precision=_HIGH, preferred_element_type=_F32)


def _proj_body(x_ref, w_ref, b_ref, a_ref, bo_ref, n_ref):
    x = x_ref[...]
    p = _dot(x, w_ref[...]) + b_ref[...]
    a_ref[...] = p[:, :DHP]
    bo_ref[...] = p[:, DHP:]
    n_ref[...] = jnp.sum(x * x, axis=1, keepdims=True)


def _proj(x, w, b):
    n, d = x.shape
    m = w.shape[1]
    blk = 512
    return pl.pallas_call(
        _proj_body,
        grid=(n // blk,),
        in_specs=[
            pl.BlockSpec((blk, d), lambda i: (i, 0)),
            pl.BlockSpec((d, m), lambda i: (0, 0)),
            pl.BlockSpec((1, m), lambda i: (0, 0)),
        ],
        out_specs=[
            pl.BlockSpec((blk, DHP), lambda i: (i, 0)),
            pl.BlockSpec((blk, DHP), lambda i: (i, 0)),
            pl.BlockSpec((blk, 1), lambda i: (i, 0)),
        ],
        out_shape=[
            jax.ShapeDtypeStruct((n, DHP), _F32),
            jax.ShapeDtypeStruct((n, DHP), _F32),
            jax.ShapeDtypeStruct((n, 1), _F32),
        ],
    )(x, w, b.reshape(1, m))


def _knn_body(t0_ref, nt_ref, xq_ref, xt_ref, x2q_ref, x2t_ref,
              lo_ref, hi_ref, nbr_ref, dscr):
    c = pl.program_id(0)
    t0 = t0_ref[c]
    nt = nt_ref[c]
    xq = xq_ref[...]
    x2q = x2q_ref[...]
    lo = lo_ref[...]
    hi = hi_ref[...]

    rowid = (lax.broadcasted_iota(jnp.int32, (R_KNN, 1), 0) + c * R_KNN)

    def fill(t, _):
        xct = xt_ref[t0 + t]
        x2c = x2t_ref[t0 + t]
        d2 = x2q - 2.0 * _dot(xq, xct) + x2c
        col = (lax.broadcasted_iota(jnp.int32, (R_KNN, C_KNN), 1)
               + (t0 + t) * C_KNN)
        valid = (col >= lo) & (col < hi) & (col != rowid)
        dscr[t] = jnp.where(valid, d2, jnp.inf)
        return 0

    lax.fori_loop(0, nt, fill, 0)

    prevd = jnp.full((R_KNN, 1), -jnp.inf, _F32)
    cols = []
    for _ in range(KSEL):
        def scan(t, carry, prevd=prevd):
            bmin, barg = carry
            dm = jnp.where(dscr[t] > prevd, dscr[t], jnp.inf)
            tmin = jnp.min(dm, axis=1, keepdims=True)
            col = (lax.broadcasted_iota(jnp.int32, (R_KNN, C_KNN), 1)
                   + (t0 + t) * C_KNN)
            targ = jnp.min(jnp.where(dm == tmin, col, _BIGI),
                           axis=1, keepdims=True)
            better = (tmin < bmin) | ((tmin == bmin) & (targ < barg))
            return (jnp.where(better, tmin, bmin),
                    jnp.where(better, targ, barg))

        bmin, barg = lax.fori_loop(
            0, nt, scan,
            (jnp.full((R_KNN, 1), jnp.inf, _F32),
             jnp.full((R_KNN, 1), _BIGI, jnp.int32)))
        prevd = bmin
        cols.append(barg)
    nbr = jnp.concatenate(cols, axis=1)
    nbr_ref[...] = jnp.clip(nbr, 0, N_PAD - 1)


def _knn(x, x2, rowlo, rowhi, t0s, nts):
    d = x.shape[1]
    xt_tiles = x.T.reshape(d, T_KNN, C_KNN).transpose(1, 0, 2)
    x2t_tiles = x2.reshape(T_KNN, 1, C_KNN)
    grid_spec = pltpu.PrefetchScalarGridSpec(
        num_scalar_prefetch=2,
        grid=(N_PAD // R_KNN,),
        in_specs=[
            pl.BlockSpec((R_KNN, d), lambda i, *_: (i, 0)),
            pl.BlockSpec((T_KNN, d, C_KNN), lambda i, *_: (0, 0, 0)),
            pl.BlockSpec((R_KNN, 1), lambda i, *_: (i, 0)),
            pl.BlockSpec((T_KNN, 1, C_KNN), lambda i, *_: (0, 0, 0)),
            pl.BlockSpec((R_KNN, 1), lambda i, *_: (i, 0)),
            pl.BlockSpec((R_KNN, 1), lambda i, *_: (i, 0)),
        ],
        out_specs=pl.BlockSpec((R_KNN, KSEL), lambda i, *_: (i, 0)),
        scratch_shapes=[pltpu.VMEM((T_KNN, R_KNN, C_KNN), _F32)],
    )
    return pl.pallas_call(
        _knn_body,
        grid_spec=grid_spec,
        out_shape=jax.ShapeDtypeStruct((N_PAD, KSEL), jnp.int32),
    )(t0s, nts, x, xt_tiles, x2, x2t_tiles, rowlo, rowhi)


_E_TOT = N_PAD * KSEL
_NW = 32
_E_PER_W = _E_TOT // _NW
_CH = 112
_NCH = _E_PER_W // _CH


def _sc_gather(table, idx):
    d = table.shape[1]
    mesh = plsc.VectorSubcoreMesh(core_axis_name="c", subcore_axis_name="s")

    @functools.partial(
        pl.kernel, mesh=mesh,
        out_type=jax.ShapeDtypeStruct((_E_TOT, d), _F32),
        scratch_types=[
            pltpu.VMEM((_E_PER_W,), jnp.int32),
            pltpu.VMEM((_CH, d), _F32),
            pltpu.VMEM((_CH, d), _F32),
            pltpu.SemaphoreType.DMA,
            pltpu.SemaphoreType.DMA,
            pltpu.SemaphoreType.DMA,
            pltpu.SemaphoreType.DMA,
        ],
    )
    def k(table_hbm, idx_hbm, out_hbm, idx_v, r0, r1, sg0, sg1, so0, so1):
        wid = lax.axis_index("s") * 2 + lax.axis_index("c")
        base = wid * _E_PER_W
        pltpu.sync_copy(idx_hbm.at[pl.ds(base, _E_PER_W)], idx_v)

        rows = (r0, r1)
        sg = (sg0, sg1)
        so = (so0, so1)

        def start_gather(c):
            return pltpu.async_copy(
                table_hbm.at[idx_v.at[pl.ds(c * _CH, _CH)]],
                rows[c % 2], sg[c % 2])

        def start_out(c):
            return pltpu.async_copy(
                rows[c % 2], out_hbm.at[pl.ds(base + c * _CH, _CH)],
                so[c % 2])

        gh = {0: start_gather(0)}
        oh = {}
        for c in range(_NCH):
            if c + 1 < _NCH:
                if c >= 1:
                    oh[c - 1].wait()
                gh[c + 1] = start_gather(c + 1)
            gh[c].wait()
            oh[c] = start_out(c)
        oh[_NCH - 2].wait()
        oh[_NCH - 1].wait()

    return k(table, idx)


def _edge_body(a_ref, b_ref, g_ref, w2_ref, b2_ref, o_ref):
    a = a_ref[...]
    w2 = w2_ref[...]
    acc = _dot(_lrelu(a + b_ref[...]), w2)
    for k in range(KSEL):
        h1 = _lrelu(a + g_ref[k])
        e = _dot(h1, w2)
        acc = jnp.maximum(acc, e)
    o_ref[...] = _lrelu(acc + b2_ref[...])


def _edge(a, b, gath, w2, b2):
    blk = 128
    return pl.pallas_call(
        _edge_body,
        grid=(N_PAD // blk,),
        in_specs=[
            pl.BlockSpec((blk, DHP), lambda i: (i, 0)),
            pl.BlockSpec((blk, DHP), lambda i: (i, 0)),
            pl.BlockSpec((KSEL, blk, DHP), lambda i: (0, i, 0)),
            pl.BlockSpec((DHP, DO), lambda i: (0, 0)),
            pl.BlockSpec((1, DO), lambda i: (0, 0)),
        ],
        out_specs=pl.BlockSpec((blk, DO), lambda i: (i, 0)),
        out_shape=jax.ShapeDtypeStruct((N_PAD, DO), _F32),
    )(a, b, gath, w2, b2.reshape(1, DO))


def _m1pool_body(h_ref, m_ref, w1_ref, b1_ref, w2_ref, b2_ref, o_ref):
    @pl.when(pl.program_id(0) == 0)
    def _():
        o_ref[...] = jnp.zeros_like(o_ref)

    h1 = _lrelu(_dot(h_ref[...], w1_ref[...]) + b1_ref[...])
    h2 = _lrelu(_dot(h1, w2_ref[...]) + b2_ref[...])
    o_ref[...] += _dot(m_ref[...], h2)


def _m1pool(h, mn, w1, b1, w2, b2):
    blk = 512
    di, dh = w1.shape
    do = w2.shape[1]
    return pl.pallas_call(
        _m1pool_body,
        grid=(N_PAD // blk,),
        in_specs=[
            pl.BlockSpec((blk, di), lambda i: (i, 0)),
            pl.BlockSpec((G_SEG, blk), lambda i: (0, i)),
            pl.BlockSpec((di, dh), lambda i: (0, 0)),
            pl.BlockSpec((1, dh), lambda i: (0, 0)),
            pl.BlockSpec((dh, do), lambda i: (0, 0)),
            pl.BlockSpec((1, do), lambda i: (0, 0)),
        ],
        out_specs=pl.BlockSpec((G_SEG, do), lambda i: (0, 0)),
        out_shape=jax.ShapeDtypeStruct((G_SEG, do), _F32),
    )(h, mn, w1, b1.reshape(1, dh), w2, b2.reshape(1, do))


def _m2_body(p_ref, w1_ref, b1_ref, w2_ref, b2_ref, o_ref):
    h = _lrelu(_dot(p_ref[...], w1_ref[...]) + b1_ref[...])
    o_ref[...] = _dot(h, w2_ref[...]) + b2_ref[...]


def _m2(pooled, w1, b1, w2, b2):
    di, dh = w1.shape
    do = w2.shape[1]
    return pl.pallas_call(
        _m2_body,
        out_shape=jax.ShapeDtypeStruct((G_SEG, do), _F32),
    )(pooled, w1, b1.reshape(1, dh), w2, b2.reshape(1, do))


def _layer(xc, rowlo, rowhi, t0s, nts, p, pre):
    d = xc.shape[1]
    w1 = p[pre + "_W1"]
    zc = jnp.zeros((d, DHP - DH), _F32)
    w1a, w1b = w1[:d], w1[d:]
    wc = jnp.concatenate([w1a - w1b, zc, w1b, zc], axis=1)
    bc = jnp.concatenate([p[pre + "_b1"],
                          jnp.zeros((2 * DHP - DH,), _F32)])
    a, b, x2 = _proj(xc, wc, bc)
    nbr = _knn(xc, x2, rowlo, rowhi, t0s, nts)
    gath = _sc_gather(b, nbr.T.reshape(-1))
    w2p = jnp.pad(p[pre + "_W2"], ((0, DHP - DH), (0, 0)))
    return _edge(a, b, gath.reshape(KSEL, N_PAD, DHP), w2p, p[pre + "_b2"])


def kernel(x, edge_index, batch, params):
    del edge_index
    n = x.shape[0]
    pad = N_PAD - n
    xp = jnp.pad(x.astype(_F32), ((0, pad), (0, 0)))
    bp = jnp.concatenate([batch.astype(jnp.int32),
                          jnp.full((pad,), G_SEG, jnp.int32)])

    starts = jnp.searchsorted(bp, jnp.arange(G_SEG + 2, dtype=jnp.int32),
                              side="left").astype(jnp.int32)
    rowlo = starts[bp][:, None]
    rowhi = starts[bp + 1][:, None]
    b2d = bp.reshape(N_PAD // R_KNN, R_KNN)
    g0, g1 = b2d[:, 0], b2d[:, -1]
    col_lo = starts[g0]
    col_hi = starts[g1 + 1]
    t0s = col_lo // C_KNN
    nts = (col_hi + C_KNN - 1) // C_KNN - t0s

    p = params
    x1 = _layer(xp, rowlo, rowhi, t0s, nts, p, "c1")
    x2 = _layer(x1, rowlo, rowhi, t0s, nts, p, "c2")
    x3 = _layer(x2, rowlo, rowhi, t0s, nts, p, "c3")
    x4 = _layer(x3, rowlo, rowhi, t0s, nts, p, "c4")

    h = jnp.concatenate([xp, x1, x2, x3, x4], axis=1)
    m = (bp[None, :] == jnp.arange(G_SEG, dtype=jnp.int32)[:, None]
         ).astype(_F32)
    cnt = jnp.sum(m, axis=1)
    mn = m / jnp.maximum(cnt, 1.0)[:, None]
    pooled = _m1pool(h, mn, p["m1_W1"], p["m1_b1"], p["m1_W2"], p["m1_b2"])
    return _m2(pooled, p["m2_W1"], p["m2_b1"], p["m2_W2"], p["m2_b2"])


# speedup vs baseline: 1.1459x; 1.0082x over previous
import functools

import jax
import jax.numpy as jnp
from jax import lax
from jax.experimental import pallas as pl
from jax.experimental.pallas import tpu as pltpu
from jax.experimental.pallas import tpu_sc as plsc

N_PAD = 10240
G_SEG = 16
KNB = 8
KSEL = 7
DH = 336
DHP = 384
DO = 256
R_KNN = 256
C_KNN = 512
T_KNN = N_PAD // C_KNN
SLOPE = 0.01

_F32 = jnp.float32
_HIGH = lax.Precision.HIGHEST
_BIGI = 2**30


def _lrelu(v):
    return jnp.maximum(v, SLOPE * v)


def _dot(a, b):
    return lax.dot_general(a, b, (((1,), (0,)), ((), ())),
                           precision=_HIGH, preferred_element_type=_F32)


def _dot_t(a, b):
    return lax.dot_general(a, b, (((1,), (1,)), ((), ())),
                           precision=_HIGH, preferred_element_type=_F32)


def _proj_body(x_ref, w_ref, b_ref, a_ref, bo_ref, n_ref):
    x = x_ref[...]
    p = _dot(x, w_ref[...]) + b_ref[...]
    a_ref[...] = p[:, :DHP]
    bo_ref[...] = p[:, DHP:]
    n_ref[...] = jnp.sum(x * x, axis=1, keepdims=True)


def _proj(x, w, b):
    n, d = x.shape
    m = w.shape[1]
    blk = 512
    return pl.pallas_call(
        _proj_body,
        grid=(n // blk,),
        in_specs=[
            pl.BlockSpec((blk, d), lambda i: (i, 0)),
            pl.BlockSpec((d, m), lambda i: (0, 0)),
            pl.BlockSpec((1, m), lambda i: (0, 0)),
        ],
        out_specs=[
            pl.BlockSpec((blk, DHP), lambda i: (i, 0)),
            pl.BlockSpec((blk, DHP), lambda i: (i, 0)),
            pl.BlockSpec((blk, 1), lambda i: (i, 0)),
        ],
        out_shape=[
            jax.ShapeDtypeStruct((n, DHP), _F32),
            jax.ShapeDtypeStruct((n, DHP), _F32),
            jax.ShapeDtypeStruct((n, 1), _F32),
        ],
    )(x, w, b.reshape(1, m))


def _knn_body(t0_ref, nt_ref, xq_ref, xt_ref, x2q_ref, x2t_ref,
              lo_ref, hi_ref, nbr_ref, dscr):
    c = pl.program_id(0)
    t0 = t0_ref[c]
    nt = nt_ref[c]
    xq = xq_ref[...]
    x2q = x2q_ref[...]
    lo = lo_ref[...]
    hi = hi_ref[...]

    rowid = (lax.broadcasted_iota(jnp.int32, (R_KNN, 1), 0) + c * R_KNN)

    def fill(t, _):
        xct = xt_ref[t0 + t]
        x2c = x2t_ref[t0 + t]
        d2 = x2q - 2.0 * _dot(xq, xct) + x2c
        col = (lax.broadcasted_iota(jnp.int32, (R_KNN, C_KNN), 1)
               + (t0 + t) * C_KNN)
        valid = (col >= lo) & (col < hi) & (col != rowid)
        dscr[t] = jnp.where(valid, d2, jnp.inf)
        return 0

    lax.fori_loop(0, nt, fill, 0)

    prevd = jnp.full((R_KNN, 1), -jnp.inf, _F32)
    cols = []
    for _ in range(KSEL):
        def scan(t, carry, prevd=prevd):
            bmin, barg = carry
            dm = jnp.where(dscr[t] > prevd, dscr[t], jnp.inf)
            tmin = jnp.min(dm, axis=1, keepdims=True)
            col = (lax.broadcasted_iota(jnp.int32, (R_KNN, C_KNN), 1)
                   + (t0 + t) * C_KNN)
            targ = jnp.min(jnp.where(dm == tmin, col, _BIGI),
                           axis=1, keepdims=True)
            better = (tmin < bmin) | ((tmin == bmin) & (targ < barg))
            return (jnp.where(better, tmin, bmin),
                    jnp.where(better, targ, barg))

        bmin, barg = lax.fori_loop(
            0, nt, scan,
            (jnp.full((R_KNN, 1), jnp.inf, _F32),
             jnp.full((R_KNN, 1), _BIGI, jnp.int32)))
        prevd = bmin
        cols.append(barg)
    nbr = jnp.concatenate(cols, axis=1)
    nbr_ref[...] = jnp.clip(nbr, 0, N_PAD - 1)


def _knn(x, x2, rowlo, rowhi, t0s, nts):
    d = x.shape[1]
    xt_tiles = x.T.reshape(d, T_KNN, C_KNN).transpose(1, 0, 2)
    x2t_tiles = x2.reshape(T_KNN, 1, C_KNN)
    grid_spec = pltpu.PrefetchScalarGridSpec(
        num_scalar_prefetch=2,
        grid=(N_PAD // R_KNN,),
        in_specs=[
            pl.BlockSpec((R_KNN, d), lambda i, *_: (i, 0)),
            pl.BlockSpec((T_KNN, d, C_KNN), lambda i, *_: (0, 0, 0)),
            pl.BlockSpec((R_KNN, 1), lambda i, *_: (i, 0)),
            pl.BlockSpec((T_KNN, 1, C_KNN), lambda i, *_: (0, 0, 0)),
            pl.BlockSpec((R_KNN, 1), lambda i, *_: (i, 0)),
            pl.BlockSpec((R_KNN, 1), lambda i, *_: (i, 0)),
        ],
        out_specs=pl.BlockSpec((R_KNN, KSEL), lambda i, *_: (i, 0)),
        scratch_shapes=[pltpu.VMEM((T_KNN, R_KNN, C_KNN), _F32)],
    )
    return pl.pallas_call(
        _knn_body,
        grid_spec=grid_spec,
        out_shape=jax.ShapeDtypeStruct((N_PAD, KSEL), jnp.int32),
    )(t0s, nts, x, xt_tiles, x2, x2t_tiles, rowlo, rowhi)


_E_TOT = N_PAD * KSEL
_NW = 32
_E_PER_W = _E_TOT // _NW
_CH = 112
_NCH = _E_PER_W // _CH


def _sc_gather(table, idx):
    d = table.shape[1]
    mesh = plsc.VectorSubcoreMesh(core_axis_name="c", subcore_axis_name="s")

    @functools.partial(
        pl.kernel, mesh=mesh,
        out_type=jax.ShapeDtypeStruct((_E_TOT, d), _F32),
        scratch_types=[
            pltpu.VMEM((_E_PER_W,), jnp.int32),
            pltpu.VMEM((_CH, d), _F32),
            pltpu.VMEM((_CH, d), _F32),
            pltpu.SemaphoreType.DMA,
            pltpu.SemaphoreType.DMA,
            pltpu.SemaphoreType.DMA,
            pltpu.SemaphoreType.DMA,
        ],
    )
    def k(table_hbm, idx_hbm, out_hbm, idx_v, r0, r1, sg0, sg1, so0, so1):
        wid = lax.axis_index("s") * 2 + lax.axis_index("c")
        base = wid * _E_PER_W
        pltpu.sync_copy(idx_hbm.at[pl.ds(base, _E_PER_W)], idx_v)

        rows = (r0, r1)
        sg = (sg0, sg1)
        so = (so0, so1)

        def start_gather(c):
            return pltpu.async_copy(
                table_hbm.at[idx_v.at[pl.ds(c * _CH, _CH)]],
                rows[c % 2], sg[c % 2])

        def start_out(c):
            return pltpu.async_copy(
                rows[c % 2], out_hbm.at[pl.ds(base + c * _CH, _CH)],
                so[c % 2])

        gh = {0: start_gather(0)}
        oh = {}
        for c in range(_NCH):
            if c + 1 < _NCH:
                if c >= 1:
                    oh[c - 1].wait()
                gh[c + 1] = start_gather(c + 1)
            gh[c].wait()
            oh[c] = start_out(c)
        oh[_NCH - 2].wait()
        oh[_NCH - 1].wait()

    return k(table, idx)


def _edge_body(a_ref, b_ref, g_ref, w2_ref, b2_ref, o_ref):
    a = a_ref[...]
    w2 = w2_ref[...]
    acc = _dot(_lrelu(a + b_ref[...]), w2)
    for k in range(KSEL):
        h1 = _lrelu(a + g_ref[k])
        e = _dot(h1, w2)
        acc = jnp.maximum(acc, e)
    o_ref[...] = _lrelu(acc + b2_ref[...])


def _edge(a, b, gath, w2, b2):
    blk = 128
    return pl.pallas_call(
        _edge_body,
        grid=(N_PAD // blk,),
        in_specs=[
            pl.BlockSpec((blk, DHP), lambda i: (i, 0)),
            pl.BlockSpec((blk, DHP), lambda i: (i, 0)),
            pl.BlockSpec((KSEL, blk, DHP), lambda i: (0, i, 0)),
            pl.BlockSpec((DHP, DO), lambda i: (0, 0)),
            pl.BlockSpec((1, DO), lambda i: (0, 0)),
        ],
        out_specs=pl.BlockSpec((blk, DO), lambda i: (i, 0)),
        out_shape=jax.ShapeDtypeStruct((N_PAD, DO), _F32),
    )(a, b, gath, w2, b2.reshape(1, DO))


def _m1pool_body(h0_ref, h1r, h2r, h3r, h4r, m_ref,
                 w10, w11, w12, w13, w14, b1_ref, w2_ref, b2_ref, o_ref):
    @pl.when(pl.program_id(0) == 0)
    def _():
        o_ref[...] = jnp.zeros_like(o_ref)

    s = (_dot(h0_ref[...], w10[...]) + _dot(h1r[...], w11[...])
         + _dot(h2r[...], w12[...]) + _dot(h3r[...], w13[...])
         + _dot(h4r[...], w14[...]))
    h1 = _lrelu(s + b1_ref[...])
    h2 = _lrelu(_dot(h1, w2_ref[...]) + b2_ref[...])
    o_ref[...] += _dot(m_ref[...], h2)


def _m1pool(hs, mn, w1, b1, w2, b2):
    blk = 512
    dh = w1.shape[1]
    do = w2.shape[1]
    splits, off = [], 0
    for h in hs:
        splits.append(w1[off:off + h.shape[1]])
        off += h.shape[1]
    row_spec = [pl.BlockSpec((blk, h.shape[1]), lambda i: (i, 0)) for h in hs]
    w_spec = [pl.BlockSpec(w.shape, lambda i: (0, 0)) for w in splits]
    return pl.pallas_call(
        _m1pool_body,
        grid=(N_PAD // blk,),
        in_specs=row_spec + [pl.BlockSpec((G_SEG, blk), lambda i: (0, i))]
        + w_spec + [
            pl.BlockSpec((1, dh), lambda i: (0, 0)),
            pl.BlockSpec(w2.shape, lambda i: (0, 0)),
            pl.BlockSpec((1, do), lambda i: (0, 0)),
        ],
        out_specs=pl.BlockSpec((G_SEG, do), lambda i: (0, 0)),
        out_shape=jax.ShapeDtypeStruct((G_SEG, do), _F32),
    )(*hs, mn, *splits, b1.reshape(1, dh), w2, b2.reshape(1, do))


def _m2_body(p_ref, w1_ref, b1_ref, w2_ref, b2_ref, o_ref):
    h = _lrelu(_dot(p_ref[...], w1_ref[...]) + b1_ref[...])
    o_ref[...] = _dot(h, w2_ref[...]) + b2_ref[...]


def _m2(pooled, w1, b1, w2, b2):
    di, dh = w1.shape
    do = w2.shape[1]
    return pl.pallas_call(
        _m2_body,
        out_shape=jax.ShapeDtypeStruct((G_SEG, do), _F32),
    )(pooled, w1, b1.reshape(1, dh), w2, b2.reshape(1, do))


def _layer(xc, rowlo, rowhi, t0s, nts, p, pre):
    d = xc.shape[1]
    w1 = p[pre + "_W1"]
    zc = jnp.zeros((d, DHP - DH), _F32)
    w1a, w1b = w1[:d], w1[d:]
    wc = jnp.concatenate([w1a - w1b, zc, w1b, zc], axis=1)
    bc = jnp.concatenate([p[pre + "_b1"],
                          jnp.zeros((2 * DHP - DH,), _F32)])
    a, b, x2 = _proj(xc, wc, bc)
    nbr = _knn(xc, x2, rowlo, rowhi, t0s, nts)
    gath = _sc_gather(b, nbr.T.reshape(-1))
    w2p = jnp.pad(p[pre + "_W2"], ((0, DHP - DH), (0, 0)))
    return _edge(a, b, gath.reshape(KSEL, N_PAD, DHP), w2p, p[pre + "_b2"])


def kernel(x, edge_index, batch, params):
    del edge_index
    n = x.shape[0]
    pad = N_PAD - n
    xp = jnp.pad(x.astype(_F32), ((0, pad), (0, 0)))
    bp = jnp.concatenate([batch.astype(jnp.int32),
                          jnp.full((pad,), G_SEG, jnp.int32)])

    starts = jnp.searchsorted(bp, jnp.arange(G_SEG + 2, dtype=jnp.int32),
                              side="left").astype(jnp.int32)
    rowlo = starts[bp][:, None]
    rowhi = starts[bp + 1][:, None]
    b2d = bp.reshape(N_PAD // R_KNN, R_KNN)
    g0, g1 = b2d[:, 0], b2d[:, -1]
    col_lo = starts[g0]
    col_hi = starts[g1 + 1]
    t0s = col_lo // C_KNN
    nts = (col_hi + C_KNN - 1) // C_KNN - t0s

    p = params
    x1 = _layer(xp, rowlo, rowhi, t0s, nts, p, "c1")
    x2 = _layer(x1, rowlo, rowhi, t0s, nts, p, "c2")
    x3 = _layer(x2, rowlo, rowhi, t0s, nts, p, "c3")
    x4 = _layer(x3, rowlo, rowhi, t0s, nts, p, "c4")

    m = (bp[None, :] == jnp.arange(G_SEG, dtype=jnp.int32)[:, None]
         ).astype(_F32)
    cnt = jnp.sum(m, axis=1)
    mn = m / jnp.maximum(cnt, 1.0)[:, None]
    pooled = _m1pool([xp, x1, x2, x3, x4], mn,
                     p["m1_W1"], p["m1_b1"], p["m1_W2"], p["m1_b2"])
    return _m2(pooled, p["m2_W1"], p["m2_b1"], p["m2_W2"], p["m2_b2"])


# speedup vs baseline: 1.1757x; 1.0260x over previous
import functools

import jax
import jax.numpy as jnp
from jax import lax
from jax.experimental import pallas as pl
from jax.experimental.pallas import tpu as pltpu
from jax.experimental.pallas import tpu_sc as plsc

N_PAD = 10240
G_SEG = 16
KNB = 8
KSEL = 7
DH = 336
DHP = 384
DO = 256
R_KNN = 256
C_KNN = 512
T_KNN = N_PAD // C_KNN
SLOPE = 0.01

_F32 = jnp.float32
_HIGH = lax.Precision.HIGHEST
_BIGI = 2**30


def _lrelu(v):
    return jnp.maximum(v, SLOPE * v)


def _dot(a, b):
    return lax.dot_general(a, b, (((1,), (0,)), ((), ())),
                           precision=_HIGH, preferred_element_type=_F32)


def _dot_t(a, b):
    return lax.dot_general(a, b, (((1,), (1,)), ((), ())),
                           precision=_HIGH, preferred_element_type=_F32)


def _proj_body(x_ref, w_ref, b_ref, a_ref, bo_ref, n_ref):
    x = x_ref[...]
    p = _dot(x, w_ref[...]) + b_ref[...]
    a_ref[...] = p[:, :DHP]
    bo_ref[...] = p[:, DHP:]
    n_ref[...] = jnp.sum(x * x, axis=1, keepdims=True)


def _proj(x, w, b):
    n, d = x.shape
    m = w.shape[1]
    blk = 512
    return pl.pallas_call(
        _proj_body,
        grid=(n // blk,),
        in_specs=[
            pl.BlockSpec((blk, d), lambda i: (i, 0)),
            pl.BlockSpec((d, m), lambda i: (0, 0)),
            pl.BlockSpec((1, m), lambda i: (0, 0)),
        ],
        out_specs=[
            pl.BlockSpec((blk, DHP), lambda i: (i, 0)),
            pl.BlockSpec((blk, DHP), lambda i: (i, 0)),
            pl.BlockSpec((blk, 1), lambda i: (i, 0)),
        ],
        out_shape=[
            jax.ShapeDtypeStruct((n, DHP), _F32),
            jax.ShapeDtypeStruct((n, DHP), _F32),
            jax.ShapeDtypeStruct((n, 1), _F32),
        ],
    )(x, w, b.reshape(1, m))


def _knn_body(t0_ref, nt_ref, xq_ref, xt_ref, x2q_ref, x2t_ref,
              lo_ref, hi_ref, nbr_ref, dscr):
    c = pl.program_id(0)
    t0 = t0_ref[c]
    nt = nt_ref[c]
    xq = xq_ref[...]
    x2q = x2q_ref[...]
    lo = lo_ref[...]
    hi = hi_ref[...]

    rowid = (lax.broadcasted_iota(jnp.int32, (R_KNN, 1), 0) + c * R_KNN)

    def fill(t, _):
        xct = xt_ref[t0 + t]
        x2c = x2t_ref[t0 + t]
        d2 = x2q - 2.0 * _dot(xq, xct) + x2c
        col = (lax.broadcasted_iota(jnp.int32, (R_KNN, C_KNN), 1)
               + (t0 + t) * C_KNN)
        valid = (col >= lo) & (col < hi) & (col != rowid)
        dscr[t] = jnp.where(valid, d2, jnp.inf)
        return 0

    lax.fori_loop(0, nt, fill, 0)

    prevd = jnp.full((R_KNN, 1), -jnp.inf, _F32)
    cols = []
    for _ in range(KSEL):
        def scan(t, carry, prevd=prevd):
            bmin, barg = carry
            dm = jnp.where(dscr[t] > prevd, dscr[t], jnp.inf)
            tmin = jnp.min(dm, axis=1, keepdims=True)
            col = (lax.broadcasted_iota(jnp.int32, (R_KNN, C_KNN), 1)
                   + (t0 + t) * C_KNN)
            targ = jnp.min(jnp.where(dm == tmin, col, _BIGI),
                           axis=1, keepdims=True)
            better = (tmin < bmin) | ((tmin == bmin) & (targ < barg))
            return (jnp.where(better, tmin, bmin),
                    jnp.where(better, targ, barg))

        bmin, barg = lax.fori_loop(
            0, nt, scan,
            (jnp.full((R_KNN, 1), jnp.inf, _F32),
             jnp.full((R_KNN, 1), _BIGI, jnp.int32)))
        prevd = bmin
        cols.append(barg)
    nbr = jnp.concatenate(cols, axis=1)
    nbr_ref[...] = jnp.clip(nbr, 0, N_PAD - 1)


def _knn(x, x2, rowlo, rowhi, t0s, nts):
    d = x.shape[1]
    xt_tiles = x.T.reshape(d, T_KNN, C_KNN).transpose(1, 0, 2)
    x2t_tiles = x2.reshape(T_KNN, 1, C_KNN)
    grid_spec = pltpu.PrefetchScalarGridSpec(
        num_scalar_prefetch=2,
        grid=(N_PAD // R_KNN,),
        in_specs=[
            pl.BlockSpec((R_KNN, d), lambda i, *_: (i, 0)),
            pl.BlockSpec((T_KNN, d, C_KNN), lambda i, *_: (0, 0, 0)),
            pl.BlockSpec((R_KNN, 1), lambda i, *_: (i, 0)),
            pl.BlockSpec((T_KNN, 1, C_KNN), lambda i, *_: (0, 0, 0)),
            pl.BlockSpec((R_KNN, 1), lambda i, *_: (i, 0)),
            pl.BlockSpec((R_KNN, 1), lambda i, *_: (i, 0)),
        ],
        out_specs=pl.BlockSpec((R_KNN, KSEL), lambda i, *_: (i, 0)),
        scratch_shapes=[pltpu.VMEM((T_KNN, R_KNN, C_KNN), _F32)],
    )
    return pl.pallas_call(
        _knn_body,
        grid_spec=grid_spec,
        out_shape=jax.ShapeDtypeStruct((N_PAD, KSEL), jnp.int32),
    )(t0s, nts, x, xt_tiles, x2, x2t_tiles, rowlo, rowhi)


_E_TOT = N_PAD * KSEL
_NW = 32
_E_PER_W = _E_TOT // _NW
_CH = 112
_NCH = _E_PER_W // _CH


def _sc_gather(table, idx):
    d = table.shape[1]
    mesh = plsc.VectorSubcoreMesh(core_axis_name="c", subcore_axis_name="s")

    @functools.partial(
        pl.kernel, mesh=mesh,
        out_type=jax.ShapeDtypeStruct((_E_TOT, d), _F32),
        scratch_types=[
            pltpu.VMEM((_E_PER_W,), jnp.int32),
            pltpu.VMEM((_CH, d), _F32),
            pltpu.VMEM((_CH, d), _F32),
            pltpu.SemaphoreType.DMA,
            pltpu.SemaphoreType.DMA,
            pltpu.SemaphoreType.DMA,
            pltpu.SemaphoreType.DMA,
        ],
    )
    def k(table_hbm, idx_hbm, out_hbm, idx_v, r0, r1, sg0, sg1, so0, so1):
        wid = lax.axis_index("s") * 2 + lax.axis_index("c")
        base = wid * _E_PER_W
        pltpu.sync_copy(idx_hbm.at[pl.ds(base, _E_PER_W)], idx_v)

        rows = (r0, r1)
        sg = (sg0, sg1)
        so = (so0, so1)

        def start_gather(c):
            return pltpu.async_copy(
                table_hbm.at[idx_v.at[pl.ds(c * _CH, _CH)]],
                rows[c % 2], sg[c % 2])

        def start_out(c):
            return pltpu.async_copy(
                rows[c % 2], out_hbm.at[pl.ds(base + c * _CH, _CH)],
                so[c % 2])

        gh = {0: start_gather(0)}
        oh = {}
        for c in range(_NCH):
            if c + 1 < _NCH:
                if c >= 1:
                    oh[c - 1].wait()
                gh[c + 1] = start_gather(c + 1)
            gh[c].wait()
            oh[c] = start_out(c)
        oh[_NCH - 2].wait()
        oh[_NCH - 1].wait()

    return k(table, idx)


def _edge_body(a_ref, b_ref, g_ref, w2_ref, b2_ref, o_ref):
    a = a_ref[...]
    w2 = w2_ref[...]
    acc = _dot(_lrelu(a + b_ref[...]), w2)
    for k in range(KSEL):
        h1 = _lrelu(a + g_ref[k])
        e = _dot(h1, w2)
        acc = jnp.maximum(acc, e)
    o_ref[...] = _lrelu(acc + b2_ref[...])


def _edge(a, b, gath, w2, b2):
    blk = 256
    return pl.pallas_call(
        _edge_body,
        grid=(N_PAD // blk,),
        in_specs=[
            pl.BlockSpec((blk, DHP), lambda i: (i, 0)),
            pl.BlockSpec((blk, DHP), lambda i: (i, 0)),
            pl.BlockSpec((KSEL, blk, DHP), lambda i: (0, i, 0)),
            pl.BlockSpec((DHP, DO), lambda i: (0, 0)),
            pl.BlockSpec((1, DO), lambda i: (0, 0)),
        ],
        out_specs=pl.BlockSpec((blk, DO), lambda i: (i, 0)),
        out_shape=jax.ShapeDtypeStruct((N_PAD, DO), _F32),
    )(a, b, gath, w2, b2.reshape(1, DO))


def _m1pool_body(h0_ref, h1r, h2r, h3r, h4r, m_ref,
                 w10, w11, w12, w13, w14, b1_ref, w2_ref, b2_ref, o_ref):
    @pl.when(pl.program_id(0) == 0)
    def _():
        o_ref[...] = jnp.zeros_like(o_ref)

    s = (_dot(h0_ref[...], w10[...]) + _dot(h1r[...], w11[...])
         + _dot(h2r[...], w12[...]) + _dot(h3r[...], w13[...])
         + _dot(h4r[...], w14[...]))
    h1 = _lrelu(s + b1_ref[...])
    h2 = _lrelu(_dot(h1, w2_ref[...]) + b2_ref[...])
    o_ref[...] += _dot(m_ref[...], h2)


def _m1pool(hs, mn, w1, b1, w2, b2):
    blk = 512
    dh = w1.shape[1]
    do = w2.shape[1]
    splits, off = [], 0
    for h in hs:
        splits.append(w1[off:off + h.shape[1]])
        off += h.shape[1]
    row_spec = [pl.BlockSpec((blk, h.shape[1]), lambda i: (i, 0)) for h in hs]
    w_spec = [pl.BlockSpec(w.shape, lambda i: (0, 0)) for w in splits]
    return pl.pallas_call(
        _m1pool_body,
        grid=(N_PAD // blk,),
        in_specs=row_spec + [pl.BlockSpec((G_SEG, blk), lambda i: (0, i))]
        + w_spec + [
            pl.BlockSpec((1, dh), lambda i: (0, 0)),
            pl.BlockSpec(w2.shape, lambda i: (0, 0)),
            pl.BlockSpec((1, do), lambda i: (0, 0)),
        ],
        out_specs=pl.BlockSpec((G_SEG, do), lambda i: (0, 0)),
        out_shape=jax.ShapeDtypeStruct((G_SEG, do), _F32),
    )(*hs, mn, *splits, b1.reshape(1, dh), w2, b2.reshape(1, do))


def _m2_body(p_ref, w1_ref, b1_ref, w2_ref, b2_ref, o_ref):
    h = _lrelu(_dot(p_ref[...], w1_ref[...]) + b1_ref[...])
    o_ref[...] = _dot(h, w2_ref[...]) + b2_ref[...]


def _m2(pooled, w1, b1, w2, b2):
    di, dh = w1.shape
    do = w2.shape[1]
    return pl.pallas_call(
        _m2_body,
        out_shape=jax.ShapeDtypeStruct((G_SEG, do), _F32),
    )(pooled, w1, b1.reshape(1, dh), w2, b2.reshape(1, do))


def _layer(xc, rowlo, rowhi, t0s, nts, p, pre):
    d = xc.shape[1]
    w1 = p[pre + "_W1"]
    zc = jnp.zeros((d, DHP - DH), _F32)
    w1a, w1b = w1[:d], w1[d:]
    wc = jnp.concatenate([w1a - w1b, zc, w1b, zc], axis=1)
    bc = jnp.concatenate([p[pre + "_b1"],
                          jnp.zeros((2 * DHP - DH,), _F32)])
    a, b, x2 = _proj(xc, wc, bc)
    nbr = _knn(xc, x2, rowlo, rowhi, t0s, nts)
    gath = _sc_gather(b, nbr.T.reshape(-1))
    w2p = jnp.pad(p[pre + "_W2"], ((0, DHP - DH), (0, 0)))
    return _edge(a, b, gath.reshape(KSEL, N_PAD, DHP), w2p, p[pre + "_b2"])


def kernel(x, edge_index, batch, params):
    del edge_index
    n = x.shape[0]
    pad = N_PAD - n
    xp = jnp.pad(x.astype(_F32), ((0, pad), (0, 0)))
    bp = jnp.concatenate([batch.astype(jnp.int32),
                          jnp.full((pad,), G_SEG, jnp.int32)])

    starts = jnp.searchsorted(bp, jnp.arange(G_SEG + 2, dtype=jnp.int32),
                              side="left").astype(jnp.int32)
    rowlo = starts[bp][:, None]
    rowhi = starts[bp + 1][:, None]
    b2d = bp.reshape(N_PAD // R_KNN, R_KNN)
    g0, g1 = b2d[:, 0], b2d[:, -1]
    col_lo = starts[g0]
    col_hi = starts[g1 + 1]
    t0s = col_lo // C_KNN
    nts = (col_hi + C_KNN - 1) // C_KNN - t0s

    p = params
    x1 = _layer(xp, rowlo, rowhi, t0s, nts, p, "c1")
    x2 = _layer(x1, rowlo, rowhi, t0s, nts, p, "c2")
    x3 = _layer(x2, rowlo, rowhi, t0s, nts, p, "c3")
    x4 = _layer(x3, rowlo, rowhi, t0s, nts, p, "c4")

    m = (bp[None, :] == jnp.arange(G_SEG, dtype=jnp.int32)[:, None]
         ).astype(_F32)
    cnt = jnp.sum(m, axis=1)
    mn = m / jnp.maximum(cnt, 1.0)[:, None]
    pooled = _m1pool([xp, x1, x2, x3, x4], mn,
                     p["m1_W1"], p["m1_b1"], p["m1_W2"], p["m1_b2"])
    return _m2(pooled, p["m2_W1"], p["m2_b1"], p["m2_W2"], p["m2_b2"])
